# baseline (speedup 1.0000x reference)
import functools

import jax
import jax.numpy as jnp
from jax import lax
from jax.experimental import pallas as pl
from jax.experimental.pallas import tpu as pltpu
from jax.experimental.pallas import tpu_sc as plsc

N = 10000
K = 8
E = 320000
HID = 32
OUT_CH = 32
DIM_PE = 16
NUM_GRAPHS = 8

NC = 2
NS = 16
B = 64
EP = 327680
ROWS_PAD = 10240
DUMMY = 10200
RPT = ROWS_PAD // NS
NBLK = 10
BN = N // NBLK
NCHUNK = 4

_mesh = plsc.VectorSubcoreMesh(core_axis_name="c", subcore_axis_name="s")


SB = 40


def _ring_blocks(tab, agg, ixs, ixd, bufs, gsems, ssems, nb):
    def sg(j, t):
        pltpu.async_copy(tab.at[ixs.at[j]], bufs[t], gsems[t])

    def wg(j, t):
        pltpu.make_async_copy(tab.at[ixs.at[j]], bufs[t], gsems[t]).wait()

    def ss(j, t):
        pltpu.async_copy(bufs[t], agg.at[ixd.at[j]], ssems[t], add=True)

    def ws(j, t):
        pltpu.make_async_copy(bufs[t], agg.at[ixd.at[j]], ssems[t]).wait()

    sg(0, 0)
    sg(1, 1)
    sg(2, 2)
    wg(0, 0)
    ss(0, 0)
    sg(3, 3)
    wg(1, 1)
    ss(1, 1)

    def body(jj, carry):
        j0 = 4 + 4 * jj
        for t in range(4):
            j = j0 + t
            ws(j - 4, t)
            sg(j, t)
            wg(j - 2, (t + 2) % 4)
            ss(j - 2, (t + 2) % 4)
        return carry

    lax.fori_loop(0, (nb - 4) // 4, body, 0)
    wg(nb - 2, (nb - 2) % 4)
    ss(nb - 2, (nb - 2) % 4)
    wg(nb - 1, (nb - 1) % 4)
    ss(nb - 1, (nb - 1) % 4)
    for j in range(nb - 4, nb):
        ws(j, j % 4)


def _edge_pass(tab, agg, srcv, dstv, row, idx_s, idx_d, bufs, gsems, ssems,
               isem_s, isem_d, nb):
    nst = nb // SB

    def istart(st, t):
        pltpu.async_copy(srcv.at[pl.ds(row + st * SB, SB)], idx_s.at[t], isem_s)
        pltpu.async_copy(dstv.at[pl.ds(row + st * SB, SB)], idx_d.at[t], isem_d)

    def iwait(st, t):
        pltpu.make_async_copy(srcv.at[pl.ds(row + st * SB, SB)], idx_s.at[t],
                              isem_s).wait()
        pltpu.make_async_copy(dstv.at[pl.ds(row + st * SB, SB)], idx_d.at[t],
                              isem_d).wait()

    istart(0, 0)
    for st in range(nst):
        t = st % 2
        iwait(st, t)
        if st + 1 < nst:
            istart(st + 1, 1 - t)
        _ring_blocks(tab, agg, idx_s.at[t], idx_d.at[t], bufs, gsems, ssems, SB)


@functools.partial(
    pl.kernel, mesh=_mesh,
    compiler_params=pltpu.CompilerParams(use_tc_tiling_on_sc=False),
    out_type=jax.ShapeDtypeStruct((NC, ROWS_PAD, 16), jnp.float32),
    scratch_types=[
        pltpu.VMEM_SHARED((ROWS_PAD, 16), jnp.float32),
        pltpu.VMEM((2, SB, B), jnp.int32),
        pltpu.VMEM((2, SB, B), jnp.int32),
        [pltpu.VMEM((B, 16), jnp.float32)] * 4,
        [pltpu.SemaphoreType.DMA] * 4,
        [pltpu.SemaphoreType.DMA] * 4,
        pltpu.SemaphoreType.DMA,
        pltpu.SemaphoreType.DMA,
    ],
)
def _sc_agg0(tab, src2, dst2, zeros16, out, agg, idx_s, idx_d, bufs,
             gsems, ssems, isem_s, isem_d):
    c = lax.axis_index("c")
    s = lax.axis_index("s")
    pltpu.sync_copy(zeros16, agg.at[pl.ds(s * RPT, RPT)])
    plsc.subcore_barrier()
    nb = EP // (NC * NS * B)
    row = (c * NS + s) * nb
    _edge_pass(tab, agg, src2, dst2, row, idx_s, idx_d, bufs,
               gsems, ssems, isem_s, isem_d, nb)
    plsc.subcore_barrier()
    pltpu.sync_copy(agg.at[pl.ds(s * RPT, RPT)],
                    out.at[c, pl.ds(s * RPT, RPT)])


@functools.partial(
    pl.kernel, mesh=_mesh,
    compiler_params=pltpu.CompilerParams(use_tc_tiling_on_sc=False),
    out_type=jax.ShapeDtypeStruct((NCHUNK, ROWS_PAD, 128), jnp.float32),
    scratch_types=[
        pltpu.VMEM_SHARED((ROWS_PAD, 128), jnp.float32),
        pltpu.VMEM((2, SB, B), jnp.int32),
        pltpu.VMEM((2, SB, B), jnp.int32),
        [pltpu.VMEM((B, 128), jnp.float32)] * 4,
        [pltpu.SemaphoreType.DMA] * 4,
        [pltpu.SemaphoreType.DMA] * 4,
        pltpu.SemaphoreType.DMA,
        pltpu.SemaphoreType.DMA,
    ],
)
def _sc_agg(tab, src_all, dst2, zeros, out, agg, idx_s, idx_d, bufs,
            gsems, ssems, isem_s, isem_d):
    c = lax.axis_index("c")
    s = lax.axis_index("s")
    nb = EP // (NS * B)
    row = s * nb

    for cc in range(2):
        chunk = c * 2 + cc
        pltpu.sync_copy(zeros, agg.at[pl.ds(s * RPT, RPT)])
        plsc.subcore_barrier()
        _edge_pass(tab, agg, src_all.at[chunk], dst2, row, idx_s, idx_d,
                   bufs, gsems, ssems, isem_s, isem_d, nb)
        plsc.subcore_barrier()
        pltpu.sync_copy(agg.at[pl.ds(s * RPT, RPT)],
                        out.at[chunk, pl.ds(s * RPT, RPT)])
        plsc.subcore_barrier()


def _tc1_body(x16, a0, W0a, b0a, W0b, b0b, out):
    z = x16[...] + a0[0] + a0[1]
    for si, sgn in enumerate((1.0, -1.0)):
        for k in range(K):
            zk = z[:, k:k + 1]
            m = jnp.maximum(sgn * zk * W0a[...] + b0a[...], 0.0)
            h = jnp.dot(m, W0b[...], preferred_element_type=jnp.float32) + b0b[...]
            g = si * K + k
            out[g // 4, :, (g % 4) * 32:(g % 4) * 32 + 32] = h


def _tc_mid_body(h, a, Wa, ba, Wb, bb, out):
    for g in range(16):
        ch, off = g // 4, (g % 4) * 32
        z = h[ch, :, off:off + 32] + a[ch, :, off:off + 32]
        m = jnp.maximum(jnp.dot(z, Wa[...], preferred_element_type=jnp.float32) + ba[...], 0.0)
        out[ch, :, off:off + 32] = (
            jnp.dot(m, Wb[...], preferred_element_type=jnp.float32) + bb[...])


def _tc3_body(h, a, W2a, b2a, W2b, b2b, Wr1, br1, Wr2, br2, bi, bip, out):
    counts = [jnp.sum(jnp.where(bip[...] == g, 1.0, 0.0)) for g in range(NUM_GRAPHS)]
    b = bi[...]
    npn = jnp.zeros_like(b)
    for g in range(NUM_GRAPHS):
        npn = npn + jnp.where(b == g, counts[g], 0.0)
    acc = jnp.zeros((h.shape[1], 32), jnp.float32)
    for k in range(K):
        hk = jnp.zeros((h.shape[1], 32), jnp.float32)
        for si in range(2):
            g = si * K + k
            ch, off = g // 4, (g % 4) * 32
            z = h[ch, :, off:off + 32] + a[ch, :, off:off + 32]
            m = jnp.maximum(jnp.dot(z, W2a[...], preferred_element_type=jnp.float32) + b2a[...], 0.0)
            hk = hk + jnp.dot(m, W2b[...], preferred_element_type=jnp.float32) + b2b[...]
        acc = acc + hk * jnp.where(npn > k, 1.0, 0.0)
    m = jnp.maximum(jnp.dot(acc, Wr1[...], preferred_element_type=jnp.float32) + br1[...], 0.0)
    out[...] = jnp.dot(m, Wr2[...], preferred_element_type=jnp.float32) + br2[...]


def _wspec(shape):
    return pl.BlockSpec(shape, lambda b: tuple(0 for _ in shape))


def _tc1(x16, agg0, W0a, b0a, W0b, b0b):
    return pl.pallas_call(
        _tc1_body,
        grid=(NBLK,),
        in_specs=[
            pl.BlockSpec((BN, 16), lambda b: (b, 0)),
            pl.BlockSpec((NC, BN, 16), lambda b: (0, b, 0)),
            _wspec((1, 32)), _wspec((1, 32)), _wspec((32, 32)), _wspec((1, 32)),
        ],
        out_specs=pl.BlockSpec((NCHUNK, BN, 128), lambda b: (0, b, 0)),
        out_shape=jax.ShapeDtypeStruct((NCHUNK, N, 128), jnp.float32),
    )(x16, agg0, W0a, b0a, W0b, b0b)


def _tc_mid(h, a, Wa, ba, Wb, bb):
    return pl.pallas_call(
        _tc_mid_body,
        grid=(NBLK,),
        in_specs=[
            pl.BlockSpec((NCHUNK, BN, 128), lambda b: (0, b, 0)),
            pl.BlockSpec((NCHUNK, BN, 128), lambda b: (0, b, 0)),
            _wspec((32, 32)), _wspec((1, 32)), _wspec((32, 32)), _wspec((1, 32)),
        ],
        out_specs=pl.BlockSpec((NCHUNK, BN, 128), lambda b: (0, b, 0)),
        out_shape=jax.ShapeDtypeStruct((NCHUNK, N, 128), jnp.float32),
    )(h, a, Wa, ba, Wb, bb)


def _tc3(h, a, W2a, b2a, W2b, b2b, Wr1, br1, Wr2, br2, bi, bip):
    return pl.pallas_call(
        _tc3_body,
        grid=(NBLK,),
        in_specs=[
            pl.BlockSpec((NCHUNK, BN, 128), lambda b: (0, b, 0)),
            pl.BlockSpec((NCHUNK, BN, 128), lambda b: (0, b, 0)),
            _wspec((32, 32)), _wspec((1, 32)), _wspec((32, 32)), _wspec((1, 32)),
            _wspec((32, 32)), _wspec((1, 32)), _wspec((32, 16)), _wspec((1, 16)),
            pl.BlockSpec((BN, 1), lambda b: (b, 0)),
            _wspec((80, 128)),
        ],
        out_specs=pl.BlockSpec((BN, DIM_PE), lambda b: (b, 0)),
        out_shape=jax.ShapeDtypeStruct((N, DIM_PE), jnp.float32),
    )(h, a, W2a, b2a, W2b, b2b, Wr1, br1, Wr2, br2, bi, bip)


@jax.jit
def kernel(x, edge_index, batch_index, W0a, b0a, W0b, b0b, W1a, b1a, W1b, b1b,
           W2a, b2a, W2b, b2b, Wr1, br1, Wr2, br2):
    x16 = jnp.pad(x[:, :, 0], ((0, 0), (0, 8)))
    pad = EP - E
    srcp = jnp.concatenate([edge_index[0], jnp.zeros((pad,), jnp.int32)])
    dstp = jnp.concatenate([edge_index[1], jnp.full((pad,), DUMMY, jnp.int32)])
    srcp2 = srcp.reshape(EP // B, B)
    dstp2 = dstp.reshape(EP // B, B)
    src_all = srcp2[None] + (jnp.arange(NCHUNK, dtype=jnp.int32) * N)[:, None, None]
    zeros = jnp.zeros((RPT, 128), jnp.float32)
    zeros16 = jnp.zeros((RPT, 16), jnp.float32)
    bi_f = batch_index.astype(jnp.float32)[:, None]
    bip = jnp.pad(bi_f[:, 0], (0, 80 * 128 - N),
                  constant_values=1e9).reshape(80, 128)
    b0a2, b0b2 = b0a[None, :], b0b[None, :]
    b1a2, b1b2 = b1a[None, :], b1b[None, :]
    b2a2, b2b2 = b2a[None, :], b2b[None, :]
    br12, br22 = br1[None, :], br2[None, :]

    agg0 = _sc_agg0(x16, srcp2, dstp2, zeros16)
    h1 = _tc1(x16, agg0, W0a, b0a2, W0b, b0b2)

    a1 = _sc_agg(h1.reshape(NCHUNK * N, 128), src_all, dstp2, zeros)
    h2 = _tc_mid(h1, a1, W1a, b1a2, W1b, b1b2)

    a2 = _sc_agg(h2.reshape(NCHUNK * N, 128), src_all, dstp2, zeros)
    return _tc3(h2, a2, W2a, b2a2, W2b, b2b2, Wr1, br12, Wr2, br22, bi_f, bip)


# speedup vs baseline: 1.0150x; 1.0150x over previous
import functools

import jax
import jax.numpy as jnp
from jax import lax
from jax.experimental import pallas as pl
from jax.experimental.pallas import tpu as pltpu
from jax.experimental.pallas import tpu_sc as plsc

N = 10000
K = 8
E = 320000
HID = 32
OUT_CH = 32
DIM_PE = 16
NUM_GRAPHS = 8

NC = 2
NS = 16
B = 128
EP = 327680
ROWS_PAD = 10240
DUMMY = 10200
RPT = ROWS_PAD // NS
NBLK = 10
BN = N // NBLK
NCHUNK = 4

_mesh = plsc.VectorSubcoreMesh(core_axis_name="c", subcore_axis_name="s")


SB = 16


def _ring_blocks(tab, agg, ixs, ixd, bufs, gsems, ssems, nb):
    def sg(j, t):
        pltpu.async_copy(tab.at[ixs.at[j]], bufs[t], gsems[t])

    def wg(j, t):
        pltpu.make_async_copy(tab.at[ixs.at[j]], bufs[t], gsems[t]).wait()

    def ss(j, t):
        pltpu.async_copy(bufs[t], agg.at[ixd.at[j]], ssems[t], add=True)

    def ws(j, t):
        pltpu.make_async_copy(bufs[t], agg.at[ixd.at[j]], ssems[t]).wait()

    sg(0, 0)
    sg(1, 1)
    wg(0, 0)
    ss(0, 0)

    def body(jj, carry):
        j0 = 2 + 2 * jj
        for t in range(2):
            j = j0 + t
            ws(j - 2, t)
            sg(j, t)
            wg(j - 1, 1 - t)
            ss(j - 1, 1 - t)
        return carry

    lax.fori_loop(0, (nb - 2) // 2, body, 0)
    wg(nb - 1, (nb - 1) % 2)
    ss(nb - 1, (nb - 1) % 2)
    ws(nb - 2, (nb - 2) % 2)
    ws(nb - 1, (nb - 1) % 2)


def _edge_pass(tab, agg, srcv, dstv, row, idx_s, idx_d, bufs, gsems, ssems,
               isem_s, isem_d, nb):
    nst = nb // SB

    def istart(st, t):
        pltpu.async_copy(srcv.at[pl.ds(row + st * SB, SB)], idx_s.at[t], isem_s)
        pltpu.async_copy(dstv.at[pl.ds(row + st * SB, SB)], idx_d.at[t], isem_d)

    def iwait(st, t):
        pltpu.make_async_copy(srcv.at[pl.ds(row + st * SB, SB)], idx_s.at[t],
                              isem_s).wait()
        pltpu.make_async_copy(dstv.at[pl.ds(row + st * SB, SB)], idx_d.at[t],
                              isem_d).wait()

    istart(0, 0)
    for st in range(nst):
        t = st % 2
        iwait(st, t)
        if st + 1 < nst:
            istart(st + 1, 1 - t)
        _ring_blocks(tab, agg, idx_s.at[t], idx_d.at[t], bufs, gsems, ssems, SB)


@functools.partial(
    pl.kernel, mesh=_mesh,
    compiler_params=pltpu.CompilerParams(use_tc_tiling_on_sc=False),
    out_type=jax.ShapeDtypeStruct((NC, ROWS_PAD, 16), jnp.float32),
    scratch_types=[
        pltpu.VMEM_SHARED((ROWS_PAD, 16), jnp.float32),
        pltpu.VMEM((2, SB, B), jnp.int32),
        pltpu.VMEM((2, SB, B), jnp.int32),
        [pltpu.VMEM((B, 16), jnp.float32)] * 2,
        [pltpu.SemaphoreType.DMA] * 2,
        [pltpu.SemaphoreType.DMA] * 2,
        pltpu.SemaphoreType.DMA,
        pltpu.SemaphoreType.DMA,
    ],
)
def _sc_agg0(tab, src2, dst2, zeros16, out, agg, idx_s, idx_d, bufs,
             gsems, ssems, isem_s, isem_d):
    c = lax.axis_index("c")
    s = lax.axis_index("s")
    pltpu.sync_copy(zeros16, agg.at[pl.ds(s * RPT, RPT)])
    plsc.subcore_barrier()
    nb = EP // (NC * NS * B)
    row = (c * NS + s) * nb
    _edge_pass(tab, agg, src2, dst2, row, idx_s, idx_d, bufs,
               gsems, ssems, isem_s, isem_d, nb)
    plsc.subcore_barrier()
    pltpu.sync_copy(agg.at[pl.ds(s * RPT, RPT)],
                    out.at[c, pl.ds(s * RPT, RPT)])


@functools.partial(
    pl.kernel, mesh=_mesh,
    compiler_params=pltpu.CompilerParams(use_tc_tiling_on_sc=False),
    out_type=jax.ShapeDtypeStruct((NCHUNK, ROWS_PAD, 128), jnp.float32),
    scratch_types=[
        pltpu.VMEM_SHARED((ROWS_PAD, 128), jnp.float32),
        pltpu.VMEM((2, SB, B), jnp.int32),
        pltpu.VMEM((2, SB, B), jnp.int32),
        [pltpu.VMEM((B, 128), jnp.float32)] * 2,
        [pltpu.SemaphoreType.DMA] * 2,
        [pltpu.SemaphoreType.DMA] * 2,
        pltpu.SemaphoreType.DMA,
        pltpu.SemaphoreType.DMA,
    ],
)
def _sc_agg(tab, src_all, dst2, zeros, out, agg, idx_s, idx_d, bufs,
            gsems, ssems, isem_s, isem_d):
    c = lax.axis_index("c")
    s = lax.axis_index("s")
    nb = EP // (NS * B)
    row = s * nb

    for cc in range(2):
        chunk = c * 2 + cc
        pltpu.sync_copy(zeros, agg.at[pl.ds(s * RPT, RPT)])
        plsc.subcore_barrier()
        _edge_pass(tab, agg, src_all.at[chunk], dst2, row, idx_s, idx_d,
                   bufs, gsems, ssems, isem_s, isem_d, nb)
        plsc.subcore_barrier()
        pltpu.sync_copy(agg.at[pl.ds(s * RPT, RPT)],
                        out.at[chunk, pl.ds(s * RPT, RPT)])
        plsc.subcore_barrier()


def _tc1_body(x16, a0, W0a, b0a, W0b, b0b, out):
    z = x16[...] + a0[0] + a0[1]
    for si, sgn in enumerate((1.0, -1.0)):
        for k in range(K):
            zk = z[:, k:k + 1]
            m = jnp.maximum(sgn * zk * W0a[...] + b0a[...], 0.0)
            h = jnp.dot(m, W0b[...], preferred_element_type=jnp.float32) + b0b[...]
            g = si * K + k
            out[g // 4, :, (g % 4) * 32:(g % 4) * 32 + 32] = h


def _tc_mid_body(h, a, Wa, ba, Wb, bb, out):
    for g in range(16):
        ch, off = g // 4, (g % 4) * 32
        z = h[ch, :, off:off + 32] + a[ch, :, off:off + 32]
        m = jnp.maximum(jnp.dot(z, Wa[...], preferred_element_type=jnp.float32) + ba[...], 0.0)
        out[ch, :, off:off + 32] = (
            jnp.dot(m, Wb[...], preferred_element_type=jnp.float32) + bb[...])


def _tc3_body(h, a, W2a, b2a, W2b, b2b, Wr1, br1, Wr2, br2, bi, bip, out):
    counts = [jnp.sum(jnp.where(bip[...] == g, 1.0, 0.0)) for g in range(NUM_GRAPHS)]
    b = bi[...]
    npn = jnp.zeros_like(b)
    for g in range(NUM_GRAPHS):
        npn = npn + jnp.where(b == g, counts[g], 0.0)
    acc = jnp.zeros((h.shape[1], 32), jnp.float32)
    for k in range(K):
        hk = jnp.zeros((h.shape[1], 32), jnp.float32)
        for si in range(2):
            g = si * K + k
            ch, off = g // 4, (g % 4) * 32
            z = h[ch, :, off:off + 32] + a[ch, :, off:off + 32]
            m = jnp.maximum(jnp.dot(z, W2a[...], preferred_element_type=jnp.float32) + b2a[...], 0.0)
            hk = hk + jnp.dot(m, W2b[...], preferred_element_type=jnp.float32) + b2b[...]
        acc = acc + hk * jnp.where(npn > k, 1.0, 0.0)
    m = jnp.maximum(jnp.dot(acc, Wr1[...], preferred_element_type=jnp.float32) + br1[...], 0.0)
    out[...] = jnp.dot(m, Wr2[...], preferred_element_type=jnp.float32) + br2[...]


def _wspec(shape):
    return pl.BlockSpec(shape, lambda b: tuple(0 for _ in shape))


def _tc1(x16, agg0, W0a, b0a, W0b, b0b):
    return pl.pallas_call(
        _tc1_body,
        grid=(NBLK,),
        in_specs=[
            pl.BlockSpec((BN, 16), lambda b: (b, 0)),
            pl.BlockSpec((NC, BN, 16), lambda b: (0, b, 0)),
            _wspec((1, 32)), _wspec((1, 32)), _wspec((32, 32)), _wspec((1, 32)),
        ],
        out_specs=pl.BlockSpec((NCHUNK, BN, 128), lambda b: (0, b, 0)),
        out_shape=jax.ShapeDtypeStruct((NCHUNK, N, 128), jnp.float32),
    )(x16, agg0, W0a, b0a, W0b, b0b)


def _tc_mid(h, a, Wa, ba, Wb, bb):
    return pl.pallas_call(
        _tc_mid_body,
        grid=(NBLK,),
        in_specs=[
            pl.BlockSpec((NCHUNK, BN, 128), lambda b: (0, b, 0)),
            pl.BlockSpec((NCHUNK, BN, 128), lambda b: (0, b, 0)),
            _wspec((32, 32)), _wspec((1, 32)), _wspec((32, 32)), _wspec((1, 32)),
        ],
        out_specs=pl.BlockSpec((NCHUNK, BN, 128), lambda b: (0, b, 0)),
        out_shape=jax.ShapeDtypeStruct((NCHUNK, N, 128), jnp.float32),
    )(h, a, Wa, ba, Wb, bb)


def _tc3(h, a, W2a, b2a, W2b, b2b, Wr1, br1, Wr2, br2, bi, bip):
    return pl.pallas_call(
        _tc3_body,
        grid=(NBLK,),
        in_specs=[
            pl.BlockSpec((NCHUNK, BN, 128), lambda b: (0, b, 0)),
            pl.BlockSpec((NCHUNK, BN, 128), lambda b: (0, b, 0)),
            _wspec((32, 32)), _wspec((1, 32)), _wspec((32, 32)), _wspec((1, 32)),
            _wspec((32, 32)), _wspec((1, 32)), _wspec((32, 16)), _wspec((1, 16)),
            pl.BlockSpec((BN, 1), lambda b: (b, 0)),
            _wspec((80, 128)),
        ],
        out_specs=pl.BlockSpec((BN, DIM_PE), lambda b: (b, 0)),
        out_shape=jax.ShapeDtypeStruct((N, DIM_PE), jnp.float32),
    )(h, a, W2a, b2a, W2b, b2b, Wr1, br1, Wr2, br2, bi, bip)


@jax.jit
def kernel(x, edge_index, batch_index, W0a, b0a, W0b, b0b, W1a, b1a, W1b, b1b,
           W2a, b2a, W2b, b2b, Wr1, br1, Wr2, br2):
    x16 = jnp.pad(x[:, :, 0], ((0, 0), (0, 8)))
    pad = EP - E
    srcp = jnp.concatenate([edge_index[0], jnp.zeros((pad,), jnp.int32)])
    dstp = jnp.concatenate([edge_index[1], jnp.full((pad,), DUMMY, jnp.int32)])
    srcp2 = srcp.reshape(EP // B, B)
    dstp2 = dstp.reshape(EP // B, B)
    src_all = srcp2[None] + (jnp.arange(NCHUNK, dtype=jnp.int32) * N)[:, None, None]
    zeros = jnp.zeros((RPT, 128), jnp.float32)
    zeros16 = jnp.zeros((RPT, 16), jnp.float32)
    bi_f = batch_index.astype(jnp.float32)[:, None]
    bip = jnp.pad(bi_f[:, 0], (0, 80 * 128 - N),
                  constant_values=1e9).reshape(80, 128)
    b0a2, b0b2 = b0a[None, :], b0b[None, :]
    b1a2, b1b2 = b1a[None, :], b1b[None, :]
    b2a2, b2b2 = b2a[None, :], b2b[None, :]
    br12, br22 = br1[None, :], br2[None, :]

    agg0 = _sc_agg0(x16, srcp2, dstp2, zeros16)
    h1 = _tc1(x16, agg0, W0a, b0a2, W0b, b0b2)

    a1 = _sc_agg(h1.reshape(NCHUNK * N, 128), src_all, dstp2, zeros)
    h2 = _tc_mid(h1, a1, W1a, b1a2, W1b, b1b2)

    a2 = _sc_agg(h2.reshape(NCHUNK * N, 128), src_all, dstp2, zeros)
    return _tc3(h2, a2, W2a, b2a2, W2b, b2b2, Wr1, br12, Wr2, br22, bi_f, bip)


# speedup vs baseline: 1.4793x; 1.4574x over previous
import functools

import jax
import jax.numpy as jnp
from jax import lax
from jax.experimental import pallas as pl
from jax.experimental.pallas import tpu as pltpu
from jax.experimental.pallas import tpu_sc as plsc

N = 10000
K = 8
E = 320000
HID = 32
OUT_CH = 32
DIM_PE = 16
NUM_GRAPHS = 8

NC = 2
NS = 16
B = 128
EP = 327680
ROWS_PAD = 10240
DUMMY = 10200
RPT = ROWS_PAD // NS
NBLK = 10
BN = N // NBLK
NCHUNK = 4

_mesh = plsc.VectorSubcoreMesh(core_axis_name="c", subcore_axis_name="s")


SB = 16


def _ring_blocks(tab, agg, ixs, ixd, bufs, gsems, ssems, nb):
    def sg(j, t):
        pltpu.async_copy(tab.at[ixs.at[j]], bufs[t], gsems[t])

    def wg(j, t):
        pltpu.make_async_copy(tab.at[ixs.at[j]], bufs[t], gsems[t]).wait()

    def ss(j, t):
        pltpu.async_copy(bufs[t], agg.at[ixd.at[j]], ssems[t], add=True)

    def ws(j, t):
        pltpu.make_async_copy(bufs[t], agg.at[ixd.at[j]], ssems[t]).wait()

    sg(0, 0)
    sg(1, 1)
    wg(0, 0)
    ss(0, 0)

    def body(jj, carry):
        j0 = 2 + 2 * jj
        for t in range(2):
            j = j0 + t
            ws(j - 2, t)
            sg(j, t)
            wg(j - 1, 1 - t)
            ss(j - 1, 1 - t)
        return carry

    lax.fori_loop(0, (nb - 2) // 2, body, 0)
    wg(nb - 1, (nb - 1) % 2)
    ss(nb - 1, (nb - 1) % 2)
    ws(nb - 2, (nb - 2) % 2)
    ws(nb - 1, (nb - 1) % 2)


def _edge_pass(tab, agg, srcv, dstv, row, idx_s, idx_d, bufs, gsems, ssems,
               isem_s, isem_d, nb):
    nst = nb // SB

    def istart(st, t):
        pltpu.async_copy(srcv.at[pl.ds(row + st * SB, SB)], idx_s.at[t], isem_s)
        pltpu.async_copy(dstv.at[pl.ds(row + st * SB, SB)], idx_d.at[t], isem_d)

    def iwait(st, t):
        pltpu.make_async_copy(srcv.at[pl.ds(row + st * SB, SB)], idx_s.at[t],
                              isem_s).wait()
        pltpu.make_async_copy(dstv.at[pl.ds(row + st * SB, SB)], idx_d.at[t],
                              isem_d).wait()

    istart(0, 0)
    for st in range(nst):
        t = st % 2
        iwait(st, t)
        if st + 1 < nst:
            istart(st + 1, 1 - t)
        _ring_blocks(tab, agg, idx_s.at[t], idx_d.at[t], bufs, gsems, ssems, SB)


@functools.partial(
    pl.kernel, mesh=_mesh,
    compiler_params=pltpu.CompilerParams(use_tc_tiling_on_sc=False),
    out_type=jax.ShapeDtypeStruct((NC, ROWS_PAD, 16), jnp.float32),
    scratch_types=[
        pltpu.VMEM_SHARED((ROWS_PAD, 16), jnp.float32),
        pltpu.VMEM((2, SB, B), jnp.int32),
        pltpu.VMEM((2, SB, B), jnp.int32),
        [pltpu.VMEM((B, 16), jnp.float32)] * 2,
        [pltpu.SemaphoreType.DMA] * 2,
        [pltpu.SemaphoreType.DMA] * 2,
        pltpu.SemaphoreType.DMA,
        pltpu.SemaphoreType.DMA,
    ],
)
def _sc_agg0(tab, src2, dst2, zeros16, out, agg, idx_s, idx_d, bufs,
             gsems, ssems, isem_s, isem_d):
    c = lax.axis_index("c")
    s = lax.axis_index("s")
    pltpu.sync_copy(zeros16, agg.at[pl.ds(s * RPT, RPT)])
    plsc.subcore_barrier()
    nb = EP // (NC * NS * B)
    row = (c * NS + s) * nb
    _edge_pass(tab, agg, src2, dst2, row, idx_s, idx_d, bufs,
               gsems, ssems, isem_s, isem_d, nb)
    plsc.subcore_barrier()
    pltpu.sync_copy(agg.at[pl.ds(s * RPT, RPT)],
                    out.at[c, pl.ds(s * RPT, RPT)])


@functools.partial(
    pl.kernel, mesh=_mesh,
    compiler_params=pltpu.CompilerParams(use_tc_tiling_on_sc=False),
    out_type=jax.ShapeDtypeStruct((NCHUNK, ROWS_PAD, 128), jnp.bfloat16),
    scratch_types=[
        pltpu.VMEM_SHARED((ROWS_PAD, 128), jnp.bfloat16),
        pltpu.VMEM((2, SB, B), jnp.int32),
        pltpu.VMEM((2, SB, B), jnp.int32),
        [pltpu.VMEM((B, 128), jnp.bfloat16)] * 2,
        [pltpu.SemaphoreType.DMA] * 2,
        [pltpu.SemaphoreType.DMA] * 2,
        pltpu.SemaphoreType.DMA,
        pltpu.SemaphoreType.DMA,
    ],
)
def _sc_agg(tab, src_all, dst2, zeros, out, agg, idx_s, idx_d, bufs,
            gsems, ssems, isem_s, isem_d):
    c = lax.axis_index("c")
    s = lax.axis_index("s")
    nb = EP // (NS * B)
    row = s * nb

    for cc in range(2):
        chunk = c * 2 + cc
        pltpu.sync_copy(zeros, agg.at[pl.ds(s * RPT, RPT)])
        plsc.subcore_barrier()
        _edge_pass(tab, agg, src_all.at[chunk], dst2, row, idx_s, idx_d,
                   bufs, gsems, ssems, isem_s, isem_d, nb)
        plsc.subcore_barrier()
        pltpu.sync_copy(agg.at[pl.ds(s * RPT, RPT)],
                        out.at[chunk, pl.ds(s * RPT, RPT)])
        plsc.subcore_barrier()


def _tc1_body(x16, a0, W0a, b0a, W0b, b0b, out):
    z = x16[...] + a0[0] + a0[1]
    for si, sgn in enumerate((1.0, -1.0)):
        for k in range(K):
            zk = z[:, k:k + 1]
            m = jnp.maximum(sgn * zk * W0a[...] + b0a[...], 0.0)
            h = jnp.dot(m, W0b[...], preferred_element_type=jnp.float32) + b0b[...]
            g = si * K + k
            out[g // 4, :, (g % 4) * 32:(g % 4) * 32 + 32] = h.astype(jnp.bfloat16)


def _tc_mid_body(h, a, Wa, ba, Wb, bb, out):
    for g in range(16):
        ch, off = g // 4, (g % 4) * 32
        z = (h[ch, :, off:off + 32].astype(jnp.float32)
             + a[ch, :, off:off + 32].astype(jnp.float32))
        m = jnp.maximum(jnp.dot(z, Wa[...], preferred_element_type=jnp.float32) + ba[...], 0.0)
        out[ch, :, off:off + 32] = (
            jnp.dot(m, Wb[...], preferred_element_type=jnp.float32) + bb[...]
        ).astype(jnp.bfloat16)


def _tc3_body(h, a, W2a, b2a, W2b, b2b, Wr1, br1, Wr2, br2, bi, bip, out):
    counts = [jnp.sum(jnp.where(bip[...] == g, 1.0, 0.0)) for g in range(NUM_GRAPHS)]
    b = bi[...]
    npn = jnp.zeros_like(b)
    for g in range(NUM_GRAPHS):
        npn = npn + jnp.where(b == g, counts[g], 0.0)
    acc = jnp.zeros((h.shape[1], 32), jnp.float32)
    for k in range(K):
        hk = jnp.zeros((h.shape[1], 32), jnp.float32)
        for si in range(2):
            g = si * K + k
            ch, off = g // 4, (g % 4) * 32
            z = (h[ch, :, off:off + 32].astype(jnp.float32)
                 + a[ch, :, off:off + 32].astype(jnp.float32))
            m = jnp.maximum(jnp.dot(z, W2a[...], preferred_element_type=jnp.float32) + b2a[...], 0.0)
            hk = hk + jnp.dot(m, W2b[...], preferred_element_type=jnp.float32) + b2b[...]
        acc = acc + hk * jnp.where(npn > k, 1.0, 0.0)
    m = jnp.maximum(jnp.dot(acc, Wr1[...], preferred_element_type=jnp.float32) + br1[...], 0.0)
    out[...] = jnp.dot(m, Wr2[...], preferred_element_type=jnp.float32) + br2[...]


def _wspec(shape):
    return pl.BlockSpec(shape, lambda b: tuple(0 for _ in shape))


def _tc1(x16, agg0, W0a, b0a, W0b, b0b):
    return pl.pallas_call(
        _tc1_body,
        grid=(NBLK,),
        in_specs=[
            pl.BlockSpec((BN, 16), lambda b: (b, 0)),
            pl.BlockSpec((NC, BN, 16), lambda b: (0, b, 0)),
            _wspec((1, 32)), _wspec((1, 32)), _wspec((32, 32)), _wspec((1, 32)),
        ],
        out_specs=pl.BlockSpec((NCHUNK, BN, 128), lambda b: (0, b, 0)),
        out_shape=jax.ShapeDtypeStruct((NCHUNK, N, 128), jnp.bfloat16),
    )(x16, agg0, W0a, b0a, W0b, b0b)


def _tc_mid(h, a, Wa, ba, Wb, bb):
    return pl.pallas_call(
        _tc_mid_body,
        grid=(NBLK,),
        in_specs=[
            pl.BlockSpec((NCHUNK, BN, 128), lambda b: (0, b, 0)),
            pl.BlockSpec((NCHUNK, BN, 128), lambda b: (0, b, 0)),
            _wspec((32, 32)), _wspec((1, 32)), _wspec((32, 32)), _wspec((1, 32)),
        ],
        out_specs=pl.BlockSpec((NCHUNK, BN, 128), lambda b: (0, b, 0)),
        out_shape=jax.ShapeDtypeStruct((NCHUNK, N, 128), jnp.bfloat16),
    )(h, a, Wa, ba, Wb, bb)


def _tc3(h, a, W2a, b2a, W2b, b2b, Wr1, br1, Wr2, br2, bi, bip):
    return pl.pallas_call(
        _tc3_body,
        grid=(NBLK,),
        in_specs=[
            pl.BlockSpec((NCHUNK, BN, 128), lambda b: (0, b, 0)),
            pl.BlockSpec((NCHUNK, BN, 128), lambda b: (0, b, 0)),
            _wspec((32, 32)), _wspec((1, 32)), _wspec((32, 32)), _wspec((1, 32)),
            _wspec((32, 32)), _wspec((1, 32)), _wspec((32, 16)), _wspec((1, 16)),
            pl.BlockSpec((BN, 1), lambda b: (b, 0)),
            _wspec((80, 128)),
        ],
        out_specs=pl.BlockSpec((BN, DIM_PE), lambda b: (b, 0)),
        out_shape=jax.ShapeDtypeStruct((N, DIM_PE), jnp.float32),
    )(h, a, W2a, b2a, W2b, b2b, Wr1, br1, Wr2, br2, bi, bip)


@jax.jit
def kernel(x, edge_index, batch_index, W0a, b0a, W0b, b0b, W1a, b1a, W1b, b1b,
           W2a, b2a, W2b, b2b, Wr1, br1, Wr2, br2):
    x16 = jnp.pad(x[:, :, 0], ((0, 0), (0, 8)))
    pad = EP - E
    srcp = jnp.concatenate([edge_index[0], jnp.zeros((pad,), jnp.int32)])
    dstp = jnp.concatenate([edge_index[1], jnp.full((pad,), DUMMY, jnp.int32)])
    srcp2 = srcp.reshape(EP // B, B)
    dstp2 = dstp.reshape(EP // B, B)
    src_all = srcp2[None] + (jnp.arange(NCHUNK, dtype=jnp.int32) * N)[:, None, None]
    zeros = jnp.zeros((RPT, 128), jnp.bfloat16)
    zeros16 = jnp.zeros((RPT, 16), jnp.float32)
    bi_f = batch_index.astype(jnp.float32)[:, None]
    bip = jnp.pad(bi_f[:, 0], (0, 80 * 128 - N),
                  constant_values=1e9).reshape(80, 128)
    b0a2, b0b2 = b0a[None, :], b0b[None, :]
    b1a2, b1b2 = b1a[None, :], b1b[None, :]
    b2a2, b2b2 = b2a[None, :], b2b[None, :]
    br12, br22 = br1[None, :], br2[None, :]

    agg0 = _sc_agg0(x16, srcp2, dstp2, zeros16)
    h1 = _tc1(x16, agg0, W0a, b0a2, W0b, b0b2)

    a1 = _sc_agg(h1.reshape(NCHUNK * N, 128), src_all, dstp2, zeros)
    h2 = _tc_mid(h1, a1, W1a, b1a2, W1b, b1b2)

    a2 = _sc_agg(h2.reshape(NCHUNK * N, 128), src_all, dstp2, zeros)
    return _tc3(h2, a2, W2a, b2a2, W2b, b2b2, Wr1, br12, Wr2, br22, bi_f, bip)


# speedup vs baseline: 1.5572x; 1.0526x over previous
import functools

import jax
import jax.numpy as jnp
from jax import lax
from jax.experimental import pallas as pl
from jax.experimental.pallas import tpu as pltpu
from jax.experimental.pallas import tpu_sc as plsc

N = 10000
K = 8
E = 320000
HID = 32
OUT_CH = 32
DIM_PE = 16
NUM_GRAPHS = 8

NC = 2
NS = 16
B = 128
EP = 327680
ROWS_PAD = 10240
DUMMY = 10200
RPT = ROWS_PAD // NS
NBLK = 10
BN = N // NBLK
NCHUNK = 4

_mesh = plsc.VectorSubcoreMesh(core_axis_name="c", subcore_axis_name="s")


SB = 40


def _ring_blocks(tab, agg, ixs, ixd, bufs, gsems, ssems, nb):
    def sg(j, t):
        pltpu.async_copy(tab.at[ixs.at[j]], bufs[t], gsems[t])

    def wg(j, t):
        pltpu.make_async_copy(tab.at[ixs.at[j]], bufs[t], gsems[t]).wait()

    def ss(j, t):
        pltpu.async_copy(bufs[t], agg.at[ixd.at[j]], ssems[t], add=True)

    def ws(j, t):
        pltpu.make_async_copy(bufs[t], agg.at[ixd.at[j]], ssems[t]).wait()

    sg(0, 0)
    sg(1, 1)
    sg(2, 2)
    wg(0, 0)
    ss(0, 0)
    sg(3, 3)
    wg(1, 1)
    ss(1, 1)

    def body(jj, carry):
        j0 = 4 + 4 * jj
        for t in range(4):
            j = j0 + t
            ws(j - 4, t)
            sg(j, t)
            wg(j - 2, (t + 2) % 4)
            ss(j - 2, (t + 2) % 4)
        return carry

    lax.fori_loop(0, (nb - 4) // 4, body, 0)
    wg(nb - 2, (nb - 2) % 4)
    ss(nb - 2, (nb - 2) % 4)
    wg(nb - 1, (nb - 1) % 4)
    ss(nb - 1, (nb - 1) % 4)
    for j in range(nb - 4, nb):
        ws(j, j % 4)


def _edge_pass(tab, agg, srcv, dstv, row, idx_s, idx_d, bufs, gsems, ssems,
               isem_s, isem_d, nb):
    nst = nb // SB

    def istart(st, t):
        pltpu.async_copy(srcv.at[pl.ds(row + st * SB, SB)], idx_s.at[t], isem_s)
        pltpu.async_copy(dstv.at[pl.ds(row + st * SB, SB)], idx_d.at[t], isem_d)

    def iwait(st, t):
        pltpu.make_async_copy(srcv.at[pl.ds(row + st * SB, SB)], idx_s.at[t],
                              isem_s).wait()
        pltpu.make_async_copy(dstv.at[pl.ds(row + st * SB, SB)], idx_d.at[t],
                              isem_d).wait()

    istart(0, 0)
    for st in range(nst):
        t = st % 2
        iwait(st, t)
        if st + 1 < nst:
            istart(st + 1, 1 - t)
        _ring_blocks(tab, agg, idx_s.at[t], idx_d.at[t], bufs, gsems, ssems, SB)


@functools.partial(
    pl.kernel, mesh=_mesh,
    compiler_params=pltpu.CompilerParams(use_tc_tiling_on_sc=False),
    out_type=jax.ShapeDtypeStruct((NC, ROWS_PAD, 16), jnp.float32),
    scratch_types=[
        pltpu.VMEM_SHARED((ROWS_PAD, 16), jnp.float32),
        pltpu.VMEM((2, SB, B), jnp.int32),
        pltpu.VMEM((2, SB, B), jnp.int32),
        [pltpu.VMEM((B, 16), jnp.float32)] * 4,
        [pltpu.SemaphoreType.DMA] * 4,
        [pltpu.SemaphoreType.DMA] * 4,
        pltpu.SemaphoreType.DMA,
        pltpu.SemaphoreType.DMA,
    ],
)
def _sc_agg0(tab, src2, dst2, zeros16, out, agg, idx_s, idx_d, bufs,
             gsems, ssems, isem_s, isem_d):
    c = lax.axis_index("c")
    s = lax.axis_index("s")
    pltpu.sync_copy(zeros16, agg.at[pl.ds(s * RPT, RPT)])
    plsc.subcore_barrier()
    nb = EP // (NC * NS * B)
    row = (c * NS + s) * nb
    _edge_pass(tab, agg, src2, dst2, row, idx_s, idx_d, bufs,
               gsems, ssems, isem_s, isem_d, nb)
    plsc.subcore_barrier()
    pltpu.sync_copy(agg.at[pl.ds(s * RPT, RPT)],
                    out.at[c, pl.ds(s * RPT, RPT)])


@functools.partial(
    pl.kernel, mesh=_mesh,
    compiler_params=pltpu.CompilerParams(use_tc_tiling_on_sc=False),
    out_type=jax.ShapeDtypeStruct((NCHUNK, ROWS_PAD, 128), jnp.bfloat16),
    scratch_types=[
        pltpu.VMEM_SHARED((ROWS_PAD, 128), jnp.bfloat16),
        pltpu.VMEM((2, SB, B), jnp.int32),
        pltpu.VMEM((2, SB, B), jnp.int32),
        [pltpu.VMEM((B, 128), jnp.bfloat16)] * 4,
        [pltpu.SemaphoreType.DMA] * 4,
        [pltpu.SemaphoreType.DMA] * 4,
        pltpu.SemaphoreType.DMA,
        pltpu.SemaphoreType.DMA,
    ],
)
def _sc_agg(tab, src_all, dst2, zeros, out, agg, idx_s, idx_d, bufs,
            gsems, ssems, isem_s, isem_d):
    c = lax.axis_index("c")
    s = lax.axis_index("s")
    nb = EP // (NS * B)
    row = s * nb

    for cc in range(2):
        chunk = c * 2 + cc
        pltpu.sync_copy(zeros, agg.at[pl.ds(s * RPT, RPT)])
        plsc.subcore_barrier()
        _edge_pass(tab, agg, src_all.at[chunk], dst2, row, idx_s, idx_d,
                   bufs, gsems, ssems, isem_s, isem_d, nb)
        plsc.subcore_barrier()
        pltpu.sync_copy(agg.at[pl.ds(s * RPT, RPT)],
                        out.at[chunk, pl.ds(s * RPT, RPT)])
        plsc.subcore_barrier()


def _tc1_body(x16, a0, W0a, b0a, W0b, b0b, out):
    z = x16[...] + a0[0] + a0[1]
    for si, sgn in enumerate((1.0, -1.0)):
        for k in range(K):
            zk = z[:, k:k + 1]
            m = jnp.maximum(sgn * zk * W0a[...] + b0a[...], 0.0)
            h = jnp.dot(m, W0b[...], preferred_element_type=jnp.float32) + b0b[...]
            g = si * K + k
            out[g // 4, :, (g % 4) * 32:(g % 4) * 32 + 32] = h.astype(jnp.bfloat16)


def _tc_mid_body(h, a, Wa, ba, Wb, bb, out):
    for g in range(16):
        ch, off = g // 4, (g % 4) * 32
        z = (h[ch, :, off:off + 32].astype(jnp.float32)
             + a[ch, :, off:off + 32].astype(jnp.float32))
        m = jnp.maximum(jnp.dot(z, Wa[...], preferred_element_type=jnp.float32) + ba[...], 0.0)
        out[ch, :, off:off + 32] = (
            jnp.dot(m, Wb[...], preferred_element_type=jnp.float32) + bb[...]
        ).astype(jnp.bfloat16)


def _tc3_body(h, a, W2a, b2a, W2b, b2b, Wr1, br1, Wr2, br2, bi, bip, out):
    counts = [jnp.sum(jnp.where(bip[...] == g, 1.0, 0.0)) for g in range(NUM_GRAPHS)]
    b = bi[...]
    npn = jnp.zeros_like(b)
    for g in range(NUM_GRAPHS):
        npn = npn + jnp.where(b == g, counts[g], 0.0)
    acc = jnp.zeros((h.shape[1], 32), jnp.float32)
    for k in range(K):
        hk = jnp.zeros((h.shape[1], 32), jnp.float32)
        for si in range(2):
            g = si * K + k
            ch, off = g // 4, (g % 4) * 32
            z = (h[ch, :, off:off + 32].astype(jnp.float32)
                 + a[ch, :, off:off + 32].astype(jnp.float32))
            m = jnp.maximum(jnp.dot(z, W2a[...], preferred_element_type=jnp.float32) + b2a[...], 0.0)
            hk = hk + jnp.dot(m, W2b[...], preferred_element_type=jnp.float32) + b2b[...]
        acc = acc + hk * jnp.where(npn > k, 1.0, 0.0)
    m = jnp.maximum(jnp.dot(acc, Wr1[...], preferred_element_type=jnp.float32) + br1[...], 0.0)
    out[...] = jnp.dot(m, Wr2[...], preferred_element_type=jnp.float32) + br2[...]


def _wspec(shape):
    return pl.BlockSpec(shape, lambda b: tuple(0 for _ in shape))


def _tc1(x16, agg0, W0a, b0a, W0b, b0b):
    return pl.pallas_call(
        _tc1_body,
        grid=(NBLK,),
        in_specs=[
            pl.BlockSpec((BN, 16), lambda b: (b, 0)),
            pl.BlockSpec((NC, BN, 16), lambda b: (0, b, 0)),
            _wspec((1, 32)), _wspec((1, 32)), _wspec((32, 32)), _wspec((1, 32)),
        ],
        out_specs=pl.BlockSpec((NCHUNK, BN, 128), lambda b: (0, b, 0)),
        out_shape=jax.ShapeDtypeStruct((NCHUNK, N, 128), jnp.bfloat16),
    )(x16, agg0, W0a, b0a, W0b, b0b)


def _tc_mid(h, a, Wa, ba, Wb, bb):
    return pl.pallas_call(
        _tc_mid_body,
        grid=(NBLK,),
        in_specs=[
            pl.BlockSpec((NCHUNK, BN, 128), lambda b: (0, b, 0)),
            pl.BlockSpec((NCHUNK, BN, 128), lambda b: (0, b, 0)),
            _wspec((32, 32)), _wspec((1, 32)), _wspec((32, 32)), _wspec((1, 32)),
        ],
        out_specs=pl.BlockSpec((NCHUNK, BN, 128), lambda b: (0, b, 0)),
        out_shape=jax.ShapeDtypeStruct((NCHUNK, N, 128), jnp.bfloat16),
    )(h, a, Wa, ba, Wb, bb)


def _tc3(h, a, W2a, b2a, W2b, b2b, Wr1, br1, Wr2, br2, bi, bip):
    return pl.pallas_call(
        _tc3_body,
        grid=(NBLK,),
        in_specs=[
            pl.BlockSpec((NCHUNK, BN, 128), lambda b: (0, b, 0)),
            pl.BlockSpec((NCHUNK, BN, 128), lambda b: (0, b, 0)),
            _wspec((32, 32)), _wspec((1, 32)), _wspec((32, 32)), _wspec((1, 32)),
            _wspec((32, 32)), _wspec((1, 32)), _wspec((32, 16)), _wspec((1, 16)),
            pl.BlockSpec((BN, 1), lambda b: (b, 0)),
            _wspec((80, 128)),
        ],
        out_specs=pl.BlockSpec((BN, DIM_PE), lambda b: (b, 0)),
        out_shape=jax.ShapeDtypeStruct((N, DIM_PE), jnp.float32),
    )(h, a, W2a, b2a, W2b, b2b, Wr1, br1, Wr2, br2, bi, bip)


@jax.jit
def kernel(x, edge_index, batch_index, W0a, b0a, W0b, b0b, W1a, b1a, W1b, b1b,
           W2a, b2a, W2b, b2b, Wr1, br1, Wr2, br2):
    x16 = jnp.pad(x[:, :, 0], ((0, 0), (0, 8)))
    pad = EP - E
    srcp = jnp.concatenate([edge_index[0], jnp.zeros((pad,), jnp.int32)])
    dstp = jnp.concatenate([edge_index[1], jnp.full((pad,), DUMMY, jnp.int32)])
    srcp2 = srcp.reshape(EP // B, B)
    dstp2 = dstp.reshape(EP // B, B)
    src_all = srcp2[None] + (jnp.arange(NCHUNK, dtype=jnp.int32) * N)[:, None, None]
    zeros = jnp.zeros((RPT, 128), jnp.bfloat16)
    zeros16 = jnp.zeros((RPT, 16), jnp.float32)
    bi_f = batch_index.astype(jnp.float32)[:, None]
    bip = jnp.pad(bi_f[:, 0], (0, 80 * 128 - N),
                  constant_values=1e9).reshape(80, 128)
    b0a2, b0b2 = b0a[None, :], b0b[None, :]
    b1a2, b1b2 = b1a[None, :], b1b[None, :]
    b2a2, b2b2 = b2a[None, :], b2b[None, :]
    br12, br22 = br1[None, :], br2[None, :]

    agg0 = _sc_agg0(x16, srcp2, dstp2, zeros16)
    h1 = _tc1(x16, agg0, W0a, b0a2, W0b, b0b2)

    a1 = _sc_agg(h1.reshape(NCHUNK * N, 128), src_all, dstp2, zeros)
    h2 = _tc_mid(h1, a1, W1a, b1a2, W1b, b1b2)

    a2 = _sc_agg(h2.reshape(NCHUNK * N, 128), src_all, dstp2, zeros)
    return _tc3(h2, a2, W2a, b2a2, W2b, b2b2, Wr1, br12, Wr2, br22, bi_f, bip)


# speedup vs baseline: 1.5650x; 1.0050x over previous
import functools

import jax
import jax.numpy as jnp
from jax import lax
from jax.experimental import pallas as pl
from jax.experimental.pallas import tpu as pltpu
from jax.experimental.pallas import tpu_sc as plsc

N = 10000
K = 8
E = 320000
HID = 32
OUT_CH = 32
DIM_PE = 16
NUM_GRAPHS = 8

NC = 2
NS = 16
B = 128
EP = 327680
ROWS_PAD = 10240
DUMMY = 10200
RPT = ROWS_PAD // NS
NBLK = 10
BN = N // NBLK
NCHUNK = 4

_mesh = plsc.VectorSubcoreMesh(core_axis_name="c", subcore_axis_name="s")


SB = 80


def _ring_blocks(tab, agg, ixs, ixd, bufs, gsems, ssems, nb):
    def sg(j, t):
        pltpu.async_copy(tab.at[ixs.at[j]], bufs[t], gsems[t])

    def wg(j, t):
        pltpu.make_async_copy(tab.at[ixs.at[j]], bufs[t], gsems[t]).wait()

    def ss(j, t):
        pltpu.async_copy(bufs[t], agg.at[ixd.at[j]], ssems[t], add=True)

    def ws(j, t):
        pltpu.make_async_copy(bufs[t], agg.at[ixd.at[j]], ssems[t]).wait()

    sg(0, 0)
    sg(1, 1)
    sg(2, 2)
    wg(0, 0)
    ss(0, 0)
    sg(3, 3)
    wg(1, 1)
    ss(1, 1)

    def body(jj, carry):
        j0 = 4 + 4 * jj
        for t in range(4):
            j = j0 + t
            ws(j - 4, t)
            sg(j, t)
            wg(j - 2, (t + 2) % 4)
            ss(j - 2, (t + 2) % 4)
        return carry

    lax.fori_loop(0, (nb - 4) // 4, body, 0)
    wg(nb - 2, (nb - 2) % 4)
    ss(nb - 2, (nb - 2) % 4)
    wg(nb - 1, (nb - 1) % 4)
    ss(nb - 1, (nb - 1) % 4)
    for j in range(nb - 4, nb):
        ws(j, j % 4)


def _edge_pass(tab, agg, srcv, dstv, row, idx_s, idx_d, bufs, gsems, ssems,
               isem_s, isem_d, nb):
    nst = nb // SB

    def istart(st, t):
        pltpu.async_copy(srcv.at[pl.ds(row + st * SB, SB)], idx_s.at[t], isem_s)
        pltpu.async_copy(dstv.at[pl.ds(row + st * SB, SB)], idx_d.at[t], isem_d)

    def iwait(st, t):
        pltpu.make_async_copy(srcv.at[pl.ds(row + st * SB, SB)], idx_s.at[t],
                              isem_s).wait()
        pltpu.make_async_copy(dstv.at[pl.ds(row + st * SB, SB)], idx_d.at[t],
                              isem_d).wait()

    istart(0, 0)
    for st in range(nst):
        t = st % 2
        iwait(st, t)
        if st + 1 < nst:
            istart(st + 1, 1 - t)
        _ring_blocks(tab, agg, idx_s.at[t], idx_d.at[t], bufs, gsems, ssems, SB)


@functools.partial(
    pl.kernel, mesh=_mesh,
    compiler_params=pltpu.CompilerParams(use_tc_tiling_on_sc=False),
    out_type=jax.ShapeDtypeStruct((NC, ROWS_PAD, 16), jnp.float32),
    scratch_types=[
        pltpu.VMEM_SHARED((ROWS_PAD, 16), jnp.float32),
        pltpu.VMEM((2, SB, B), jnp.int32),
        pltpu.VMEM((2, SB, B), jnp.int32),
        [pltpu.VMEM((B, 16), jnp.float32)] * 4,
        [pltpu.SemaphoreType.DMA] * 4,
        [pltpu.SemaphoreType.DMA] * 4,
        pltpu.SemaphoreType.DMA,
        pltpu.SemaphoreType.DMA,
    ],
)
def _sc_agg0(tab, src2, dst2, zeros16, out, agg, idx_s, idx_d, bufs,
             gsems, ssems, isem_s, isem_d):
    c = lax.axis_index("c")
    s = lax.axis_index("s")
    pltpu.sync_copy(zeros16, agg.at[pl.ds(s * RPT, RPT)])
    plsc.subcore_barrier()
    nb = EP // (NC * NS * B)
    row = (c * NS + s) * nb
    _edge_pass(tab, agg, src2, dst2, row, idx_s, idx_d, bufs,
               gsems, ssems, isem_s, isem_d, nb)
    plsc.subcore_barrier()
    pltpu.sync_copy(agg.at[pl.ds(s * RPT, RPT)],
                    out.at[c, pl.ds(s * RPT, RPT)])


@functools.partial(
    pl.kernel, mesh=_mesh,
    compiler_params=pltpu.CompilerParams(use_tc_tiling_on_sc=False),
    out_type=jax.ShapeDtypeStruct((NCHUNK, ROWS_PAD, 128), jnp.bfloat16),
    scratch_types=[
        pltpu.VMEM_SHARED((ROWS_PAD, 128), jnp.bfloat16),
        pltpu.VMEM((2, SB, B), jnp.int32),
        pltpu.VMEM((2, SB, B), jnp.int32),
        [pltpu.VMEM((B, 128), jnp.bfloat16)] * 4,
        [pltpu.SemaphoreType.DMA] * 4,
        [pltpu.SemaphoreType.DMA] * 4,
        pltpu.SemaphoreType.DMA,
        pltpu.SemaphoreType.DMA,
    ],
)
def _sc_agg(tab, src_all, dst2, zeros, out, agg, idx_s, idx_d, bufs,
            gsems, ssems, isem_s, isem_d):
    c = lax.axis_index("c")
    s = lax.axis_index("s")
    nb = EP // (NS * B)
    row = s * nb

    for cc in range(2):
        chunk = c * 2 + cc
        pltpu.sync_copy(zeros, agg.at[pl.ds(s * RPT, RPT)])
        plsc.subcore_barrier()
        _edge_pass(tab, agg, src_all.at[chunk], dst2, row, idx_s, idx_d,
                   bufs, gsems, ssems, isem_s, isem_d, nb)
        plsc.subcore_barrier()
        pltpu.sync_copy(agg.at[pl.ds(s * RPT, RPT)],
                        out.at[chunk, pl.ds(s * RPT, RPT)])
        plsc.subcore_barrier()


def _tc1_body(x16, a0, W0a, b0a, W0b, b0b, out):
    z = x16[...] + a0[0] + a0[1]
    for si, sgn in enumerate((1.0, -1.0)):
        for k in range(K):
            zk = z[:, k:k + 1]
            m = jnp.maximum(sgn * zk * W0a[...] + b0a[...], 0.0)
            h = jnp.dot(m, W0b[...], preferred_element_type=jnp.float32) + b0b[...]
            g = si * K + k
            out[g // 4, :, (g % 4) * 32:(g % 4) * 32 + 32] = h.astype(jnp.bfloat16)


def _tc_mid_body(h, a, Wa, ba, Wb, bb, out):
    for g in range(16):
        ch, off = g // 4, (g % 4) * 32
        z = (h[ch, :, off:off + 32].astype(jnp.float32)
             + a[ch, :, off:off + 32].astype(jnp.float32))
        m = jnp.maximum(jnp.dot(z, Wa[...], preferred_element_type=jnp.float32) + ba[...], 0.0)
        out[ch, :, off:off + 32] = (
            jnp.dot(m, Wb[...], preferred_element_type=jnp.float32) + bb[...]
        ).astype(jnp.bfloat16)


def _tc3_body(h, a, W2a, b2a, W2b, b2b, Wr1, br1, Wr2, br2, bi, bip, out):
    counts = [jnp.sum(jnp.where(bip[...] == g, 1.0, 0.0)) for g in range(NUM_GRAPHS)]
    b = bi[...]
    npn = jnp.zeros_like(b)
    for g in range(NUM_GRAPHS):
        npn = npn + jnp.where(b == g, counts[g], 0.0)
    acc = jnp.zeros((h.shape[1], 32), jnp.float32)
    for k in range(K):
        hk = jnp.zeros((h.shape[1], 32), jnp.float32)
        for si in range(2):
            g = si * K + k
            ch, off = g // 4, (g % 4) * 32
            z = (h[ch, :, off:off + 32].astype(jnp.float32)
                 + a[ch, :, off:off + 32].astype(jnp.float32))
            m = jnp.maximum(jnp.dot(z, W2a[...], preferred_element_type=jnp.float32) + b2a[...], 0.0)
            hk = hk + jnp.dot(m, W2b[...], preferred_element_type=jnp.float32) + b2b[...]
        acc = acc + hk * jnp.where(npn > k, 1.0, 0.0)
    m = jnp.maximum(jnp.dot(acc, Wr1[...], preferred_element_type=jnp.float32) + br1[...], 0.0)
    out[...] = jnp.dot(m, Wr2[...], preferred_element_type=jnp.float32) + br2[...]


def _wspec(shape):
    return pl.BlockSpec(shape, lambda b: tuple(0 for _ in shape))


def _tc1(x16, agg0, W0a, b0a, W0b, b0b):
    return pl.pallas_call(
        _tc1_body,
        grid=(NBLK,),
        in_specs=[
            pl.BlockSpec((BN, 16), lambda b: (b, 0)),
            pl.BlockSpec((NC, BN, 16), lambda b: (0, b, 0)),
            _wspec((1, 32)), _wspec((1, 32)), _wspec((32, 32)), _wspec((1, 32)),
        ],
        out_specs=pl.BlockSpec((NCHUNK, BN, 128), lambda b: (0, b, 0)),
        out_shape=jax.ShapeDtypeStruct((NCHUNK, N, 128), jnp.bfloat16),
    )(x16, agg0, W0a, b0a, W0b, b0b)


def _tc_mid(h, a, Wa, ba, Wb, bb):
    return pl.pallas_call(
        _tc_mid_body,
        grid=(NBLK,),
        in_specs=[
            pl.BlockSpec((NCHUNK, BN, 128), lambda b: (0, b, 0)),
            pl.BlockSpec((NCHUNK, BN, 128), lambda b: (0, b, 0)),
            _wspec((32, 32)), _wspec((1, 32)), _wspec((32, 32)), _wspec((1, 32)),
        ],
        out_specs=pl.BlockSpec((NCHUNK, BN, 128), lambda b: (0, b, 0)),
        out_shape=jax.ShapeDtypeStruct((NCHUNK, N, 128), jnp.bfloat16),
    )(h, a, Wa, ba, Wb, bb)


def _tc3(h, a, W2a, b2a, W2b, b2b, Wr1, br1, Wr2, br2, bi, bip):
    return pl.pallas_call(
        _tc3_body,
        grid=(NBLK,),
        in_specs=[
            pl.BlockSpec((NCHUNK, BN, 128), lambda b: (0, b, 0)),
            pl.BlockSpec((NCHUNK, BN, 128), lambda b: (0, b, 0)),
            _wspec((32, 32)), _wspec((1, 32)), _wspec((32, 32)), _wspec((1, 32)),
            _wspec((32, 32)), _wspec((1, 32)), _wspec((32, 16)), _wspec((1, 16)),
            pl.BlockSpec((BN, 1), lambda b: (b, 0)),
            _wspec((80, 128)),
        ],
        out_specs=pl.BlockSpec((BN, DIM_PE), lambda b: (b, 0)),
        out_shape=jax.ShapeDtypeStruct((N, DIM_PE), jnp.float32),
    )(h, a, W2a, b2a, W2b, b2b, Wr1, br1, Wr2, br2, bi, bip)


@jax.jit
def kernel(x, edge_index, batch_index, W0a, b0a, W0b, b0b, W1a, b1a, W1b, b1b,
           W2a, b2a, W2b, b2b, Wr1, br1, Wr2, br2):
    x16 = jnp.pad(x[:, :, 0], ((0, 0), (0, 8)))
    pad = EP - E
    srcp = jnp.concatenate([edge_index[0], jnp.zeros((pad,), jnp.int32)])
    dstp = jnp.concatenate([edge_index[1], jnp.full((pad,), DUMMY, jnp.int32)])
    srcp2 = srcp.reshape(EP // B, B)
    dstp2 = dstp.reshape(EP // B, B)
    src_all = srcp2[None] + (jnp.arange(NCHUNK, dtype=jnp.int32) * N)[:, None, None]
    zeros = jnp.zeros((RPT, 128), jnp.bfloat16)
    zeros16 = jnp.zeros((RPT, 16), jnp.float32)
    bi_f = batch_index.astype(jnp.float32)[:, None]
    bip = jnp.pad(bi_f[:, 0], (0, 80 * 128 - N),
                  constant_values=1e9).reshape(80, 128)
    b0a2, b0b2 = b0a[None, :], b0b[None, :]
    b1a2, b1b2 = b1a[None, :], b1b[None, :]
    b2a2, b2b2 = b2a[None, :], b2b[None, :]
    br12, br22 = br1[None, :], br2[None, :]

    agg0 = _sc_agg0(x16, srcp2, dstp2, zeros16)
    h1 = _tc1(x16, agg0, W0a, b0a2, W0b, b0b2)

    a1 = _sc_agg(h1.reshape(NCHUNK * N, 128), src_all, dstp2, zeros)
    h2 = _tc_mid(h1, a1, W1a, b1a2, W1b, b1b2)

    a2 = _sc_agg(h2.reshape(NCHUNK * N, 128), src_all, dstp2, zeros)
    return _tc3(h2, a2, W2a, b2a2, W2b, b2b2, Wr1, br12, Wr2, br22, bi_f, bip)


# speedup vs baseline: 1.5891x; 1.0155x over previous
import functools

import jax
import jax.numpy as jnp
from jax import lax
from jax.experimental import pallas as pl
from jax.experimental.pallas import tpu as pltpu
from jax.experimental.pallas import tpu_sc as plsc

N = 10000
K = 8
E = 320000
HID = 32
OUT_CH = 32
DIM_PE = 16
NUM_GRAPHS = 8

NC = 2
NS = 16
B = 128
EP = 327680
ROWS_PAD = 10240
DUMMY = 10200
RPT = ROWS_PAD // NS
NBLK = 10
BN = N // NBLK
NCHUNK = 4

_mesh = plsc.VectorSubcoreMesh(core_axis_name="c", subcore_axis_name="s")


SB = 80


def _ring_blocks(tab, agg, ixs, ixd, bufs, gsems, ssems, nb):
    def sg(j, t):
        pltpu.async_copy(tab.at[ixs.at[j]], bufs[t], gsems[t])

    def wg(j, t):
        pltpu.make_async_copy(tab.at[ixs.at[j]], bufs[t], gsems[t]).wait()

    def ss(j, t):
        pltpu.async_copy(bufs[t], agg.at[ixd.at[j]], ssems[t], add=True)

    def ws(j, t):
        pltpu.make_async_copy(bufs[t], agg.at[ixd.at[j]], ssems[t]).wait()

    sg(0, 0)
    sg(1, 1)
    sg(2, 2)
    wg(0, 0)
    ss(0, 0)
    sg(3, 3)
    wg(1, 1)
    ss(1, 1)

    def body(jj, carry):
        j0 = 4 + 4 * jj
        for t in range(4):
            j = j0 + t
            ws(j - 4, t)
            sg(j, t)
            wg(j - 2, (t + 2) % 4)
            ss(j - 2, (t + 2) % 4)
        return carry

    lax.fori_loop(0, (nb - 4) // 4, body, 0)
    wg(nb - 2, (nb - 2) % 4)
    ss(nb - 2, (nb - 2) % 4)
    wg(nb - 1, (nb - 1) % 4)
    ss(nb - 1, (nb - 1) % 4)
    for j in range(nb - 4, nb):
        ws(j, j % 4)


def _edge_pass(tab, agg, srcv, dstv, row, idx_s, idx_d, bufs, gsems, ssems,
               isem_s, isem_d, nb):
    nst = nb // SB

    def istart(st, t):
        pltpu.async_copy(srcv.at[pl.ds(row + st * SB, SB)], idx_s.at[t], isem_s)
        pltpu.async_copy(dstv.at[pl.ds(row + st * SB, SB)], idx_d.at[t], isem_d)

    def iwait(st, t):
        pltpu.make_async_copy(srcv.at[pl.ds(row + st * SB, SB)], idx_s.at[t],
                              isem_s).wait()
        pltpu.make_async_copy(dstv.at[pl.ds(row + st * SB, SB)], idx_d.at[t],
                              isem_d).wait()

    istart(0, 0)
    for st in range(nst):
        t = st % 2
        iwait(st, t)
        if st + 1 < nst:
            istart(st + 1, 1 - t)
        _ring_blocks(tab, agg, idx_s.at[t], idx_d.at[t], bufs, gsems, ssems, SB)


@functools.partial(
    pl.kernel, mesh=_mesh,
    compiler_params=pltpu.CompilerParams(use_tc_tiling_on_sc=False),
    out_type=jax.ShapeDtypeStruct((NC, ROWS_PAD, 16), jnp.float32),
    scratch_types=[
        pltpu.VMEM_SHARED((ROWS_PAD, 16), jnp.float32),
        pltpu.VMEM((2, SB, B), jnp.int32),
        pltpu.VMEM((2, SB, B), jnp.int32),
        [pltpu.VMEM((B, 16), jnp.float32)] * 4,
        [pltpu.SemaphoreType.DMA] * 4,
        [pltpu.SemaphoreType.DMA] * 4,
        pltpu.SemaphoreType.DMA,
        pltpu.SemaphoreType.DMA,
    ],
)
def _sc_agg0(tab, src2, dst2, zeros16, out, agg, idx_s, idx_d, bufs,
             gsems, ssems, isem_s, isem_d):
    c = lax.axis_index("c")
    s = lax.axis_index("s")
    pltpu.sync_copy(zeros16, agg.at[pl.ds(s * RPT, RPT)])
    plsc.subcore_barrier()
    nb = EP // (NC * NS * B)
    row = (c * NS + s) * nb
    _edge_pass(tab, agg, src2, dst2, row, idx_s, idx_d, bufs,
               gsems, ssems, isem_s, isem_d, nb)
    plsc.subcore_barrier()
    pltpu.sync_copy(agg.at[pl.ds(s * RPT, RPT)],
                    out.at[c, pl.ds(s * RPT, RPT)])


@functools.partial(
    pl.kernel, mesh=_mesh,
    compiler_params=pltpu.CompilerParams(use_tc_tiling_on_sc=False),
    out_type=jax.ShapeDtypeStruct((NCHUNK, ROWS_PAD, 128), jnp.bfloat16),
    scratch_types=[
        pltpu.VMEM_SHARED((ROWS_PAD, 128), jnp.bfloat16),
        pltpu.VMEM((2, SB, B), jnp.int32),
        pltpu.VMEM((2, SB, B), jnp.int32),
        [pltpu.VMEM((B, 128), jnp.bfloat16)] * 4,
        [pltpu.SemaphoreType.DMA] * 4,
        [pltpu.SemaphoreType.DMA] * 4,
        pltpu.SemaphoreType.DMA,
        pltpu.SemaphoreType.DMA,
    ],
)
def _sc_agg(tab, src2, dst2, zeros, out, agg, idx_s, idx_d, bufs,
            gsems, ssems, isem_s, isem_d):
    c = lax.axis_index("c")
    s = lax.axis_index("s")
    nb = EP // (NS * B)
    row = s * nb

    for cc in range(2):
        chunk = c * 2 + cc
        pltpu.sync_copy(zeros, agg.at[pl.ds(s * RPT, RPT)])
        plsc.subcore_barrier()
        _edge_pass(tab.at[chunk], agg, src2, dst2, row, idx_s, idx_d,
                   bufs, gsems, ssems, isem_s, isem_d, nb)
        plsc.subcore_barrier()
        pltpu.sync_copy(agg.at[pl.ds(s * RPT, RPT)],
                        out.at[chunk, pl.ds(s * RPT, RPT)])
        plsc.subcore_barrier()


def _tc1_body(x16, a0, W0a, b0a, W0b, b0b, out):
    z = x16[...] + a0[0] + a0[1]
    for si, sgn in enumerate((1.0, -1.0)):
        for k in range(K):
            zk = z[:, k:k + 1]
            m = jnp.maximum(sgn * zk * W0a[...] + b0a[...], 0.0)
            h = jnp.dot(m, W0b[...], preferred_element_type=jnp.float32) + b0b[...]
            g = si * K + k
            out[g // 4, :, (g % 4) * 32:(g % 4) * 32 + 32] = h.astype(jnp.bfloat16)


def _tc_mid_body(h, a, Wa, ba, Wb, bb, out):
    for g in range(16):
        ch, off = g // 4, (g % 4) * 32
        z = (h[ch, :, off:off + 32].astype(jnp.float32)
             + a[ch, :, off:off + 32].astype(jnp.float32))
        m = jnp.maximum(jnp.dot(z, Wa[...], preferred_element_type=jnp.float32) + ba[...], 0.0)
        out[ch, :, off:off + 32] = (
            jnp.dot(m, Wb[...], preferred_element_type=jnp.float32) + bb[...]
        ).astype(jnp.bfloat16)


def _tc3_body(h, a, W2a, b2a, W2b, b2b, Wr1, br1, Wr2, br2, bi, bip, out):
    counts = [jnp.sum(jnp.where(bip[...] == g, 1.0, 0.0)) for g in range(NUM_GRAPHS)]
    b = bi[...]
    npn = jnp.zeros_like(b)
    for g in range(NUM_GRAPHS):
        npn = npn + jnp.where(b == g, counts[g], 0.0)
    acc = jnp.zeros((h.shape[1], 32), jnp.float32)
    for k in range(K):
        hk = jnp.zeros((h.shape[1], 32), jnp.float32)
        for si in range(2):
            g = si * K + k
            ch, off = g // 4, (g % 4) * 32
            z = (h[ch, :, off:off + 32].astype(jnp.float32)
                 + a[ch, :, off:off + 32].astype(jnp.float32))
            m = jnp.maximum(jnp.dot(z, W2a[...], preferred_element_type=jnp.float32) + b2a[...], 0.0)
            hk = hk + jnp.dot(m, W2b[...], preferred_element_type=jnp.float32) + b2b[...]
        acc = acc + hk * jnp.where(npn > k, 1.0, 0.0)
    m = jnp.maximum(jnp.dot(acc, Wr1[...], preferred_element_type=jnp.float32) + br1[...], 0.0)
    out[...] = jnp.dot(m, Wr2[...], preferred_element_type=jnp.float32) + br2[...]


def _wspec(shape):
    return pl.BlockSpec(shape, lambda b: tuple(0 for _ in shape))


def _tc1(x16, agg0, W0a, b0a, W0b, b0b):
    return pl.pallas_call(
        _tc1_body,
        grid=(NBLK,),
        in_specs=[
            pl.BlockSpec((BN, 16), lambda b: (b, 0)),
            pl.BlockSpec((NC, BN, 16), lambda b: (0, b, 0)),
            _wspec((1, 32)), _wspec((1, 32)), _wspec((32, 32)), _wspec((1, 32)),
        ],
        out_specs=pl.BlockSpec((NCHUNK, BN, 128), lambda b: (0, b, 0)),
        out_shape=jax.ShapeDtypeStruct((NCHUNK, N, 128), jnp.bfloat16),
    )(x16, agg0, W0a, b0a, W0b, b0b)


def _tc_mid(h, a, Wa, ba, Wb, bb):
    return pl.pallas_call(
        _tc_mid_body,
        grid=(NBLK,),
        in_specs=[
            pl.BlockSpec((NCHUNK, BN, 128), lambda b: (0, b, 0)),
            pl.BlockSpec((NCHUNK, BN, 128), lambda b: (0, b, 0)),
            _wspec((32, 32)), _wspec((1, 32)), _wspec((32, 32)), _wspec((1, 32)),
        ],
        out_specs=pl.BlockSpec((NCHUNK, BN, 128), lambda b: (0, b, 0)),
        out_shape=jax.ShapeDtypeStruct((NCHUNK, N, 128), jnp.bfloat16),
    )(h, a, Wa, ba, Wb, bb)


def _tc3(h, a, W2a, b2a, W2b, b2b, Wr1, br1, Wr2, br2, bi, bip):
    return pl.pallas_call(
        _tc3_body,
        grid=(NBLK,),
        in_specs=[
            pl.BlockSpec((NCHUNK, BN, 128), lambda b: (0, b, 0)),
            pl.BlockSpec((NCHUNK, BN, 128), lambda b: (0, b, 0)),
            _wspec((32, 32)), _wspec((1, 32)), _wspec((32, 32)), _wspec((1, 32)),
            _wspec((32, 32)), _wspec((1, 32)), _wspec((32, 16)), _wspec((1, 16)),
            pl.BlockSpec((BN, 1), lambda b: (b, 0)),
            _wspec((80, 128)),
        ],
        out_specs=pl.BlockSpec((BN, DIM_PE), lambda b: (b, 0)),
        out_shape=jax.ShapeDtypeStruct((N, DIM_PE), jnp.float32),
    )(h, a, W2a, b2a, W2b, b2b, Wr1, br1, Wr2, br2, bi, bip)


@jax.jit
def kernel(x, edge_index, batch_index, W0a, b0a, W0b, b0b, W1a, b1a, W1b, b1b,
           W2a, b2a, W2b, b2b, Wr1, br1, Wr2, br2):
    x16 = jnp.pad(x[:, :, 0], ((0, 0), (0, 8)))
    pad = EP - E
    srcp = jnp.concatenate([edge_index[0], jnp.zeros((pad,), jnp.int32)])
    dstp = jnp.concatenate([edge_index[1], jnp.full((pad,), DUMMY, jnp.int32)])
    srcp2 = srcp.reshape(EP // B, B)
    dstp2 = dstp.reshape(EP // B, B)
    zeros = jnp.zeros((RPT, 128), jnp.bfloat16)
    zeros16 = jnp.zeros((RPT, 16), jnp.float32)
    bi_f = batch_index.astype(jnp.float32)[:, None]
    bip = jnp.pad(bi_f[:, 0], (0, 80 * 128 - N),
                  constant_values=1e9).reshape(80, 128)
    b0a2, b0b2 = b0a[None, :], b0b[None, :]
    b1a2, b1b2 = b1a[None, :], b1b[None, :]
    b2a2, b2b2 = b2a[None, :], b2b[None, :]
    br12, br22 = br1[None, :], br2[None, :]

    agg0 = _sc_agg0(x16, srcp2, dstp2, zeros16)
    h1 = _tc1(x16, agg0, W0a, b0a2, W0b, b0b2)

    a1 = _sc_agg(h1, srcp2, dstp2, zeros)
    h2 = _tc_mid(h1, a1, W1a, b1a2, W1b, b1b2)

    a2 = _sc_agg(h2, srcp2, dstp2, zeros)
    return _tc3(h2, a2, W2a, b2a2, W2b, b2b2, Wr1, br12, Wr2, br22, bi_f, bip)
